# baseline (speedup 1.0000x reference)
import jax
import jax.numpy as jnp
from jax.experimental import pallas as pl
from jax.experimental.pallas import tpu as pltpu

_BB = 16


def _eqv_body(x_ref, lam_ref, gam_ref, o_ref):
    bb, m, f = x_ref.shape
    o = o_ref.shape[-1]
    xb = x_ref[...]
    x2 = xb.reshape(bb * m, f)
    lmat = jax.lax.dot_general(
        x2, lam_ref[...], (((1,), (0,)), ((), ())),
        preferred_element_type=jnp.float32)
    s = jnp.sum(xb, axis=1)
    pooled = jax.lax.dot_general(
        s, gam_ref[...], (((1,), (0,)), ((), ())),
        preferred_element_type=jnp.float32)
    out = lmat.reshape(bb, m, o) - pooled[:, None, :]
    o_ref[...] = jnp.maximum(out, 0.0)


def kernel(x, lam, gam):
    b, m, f = x.shape
    o = lam.shape[1]
    bb = _BB
    return pl.pallas_call(
        _eqv_body,
        out_shape=jax.ShapeDtypeStruct((b, m, o), x.dtype),
        grid=(b // bb,),
        in_specs=[
            pl.BlockSpec((bb, m, f), lambda i: (i, 0, 0)),
            pl.BlockSpec((f, o), lambda i: (0, 0)),
            pl.BlockSpec((f, o), lambda i: (0, 0)),
        ],
        out_specs=pl.BlockSpec((bb, m, o), lambda i: (i, 0, 0)),
        compiler_params=pltpu.CompilerParams(
            dimension_semantics=("parallel",),
        ),
        name="equivariant_fused",
    )(x, lam, gam)


# speedup vs baseline: 2.5342x; 2.5342x over previous
import jax
import jax.numpy as jnp
from jax.experimental import pallas as pl
from jax.experimental.pallas import tpu as pltpu

_BB = 16


def _eqv_body(xt_ref, w_ref, o_ref):
    bb, f, m = xt_ref.shape
    w = w_ref[...]
    for j in range(bb):
        xj = xt_ref[j]
        s = jnp.sum(xj, axis=1, keepdims=True)
        sb = jnp.broadcast_to(s, (f, m))
        xaug = jnp.concatenate([xj, sb], axis=0)
        outj = jax.lax.dot_general(
            xaug, w, (((0,), (0,)), ((), ())),
            preferred_element_type=jnp.float32)
        o_ref[j] = jnp.maximum(outj, 0.0)


def kernel(x, lam, gam):
    b, m, f = x.shape
    o = lam.shape[1]
    xt = jnp.transpose(x, (0, 2, 1))
    w = jnp.concatenate([lam, -gam], axis=0)
    bb = _BB
    return pl.pallas_call(
        _eqv_body,
        out_shape=jax.ShapeDtypeStruct((b, m, o), x.dtype),
        grid=(b // bb,),
        in_specs=[
            pl.BlockSpec((bb, f, m), lambda i: (i, 0, 0)),
            pl.BlockSpec((2 * f, o), lambda i: (0, 0)),
        ],
        out_specs=pl.BlockSpec((bb, m, o), lambda i: (i, 0, 0)),
        compiler_params=pltpu.CompilerParams(
            dimension_semantics=("parallel",),
        ),
        name="equivariant_fused",
    )(xt, w)


# speedup vs baseline: 2.7132x; 1.0706x over previous
import jax
import jax.numpy as jnp
from jax.experimental import pallas as pl
from jax.experimental.pallas import tpu as pltpu

_BB = 32


def _eqv_body(xt_ref, w_ref, o_ref):
    bb, f, m = xt_ref.shape
    w = w_ref[...]
    for j in range(bb):
        xj = xt_ref[j]
        s = jnp.sum(xj, axis=1, keepdims=True)
        sb = jnp.broadcast_to(s, (f, m))
        xaug = jnp.concatenate([xj, sb], axis=0)
        outj = jax.lax.dot_general(
            xaug, w, (((0,), (0,)), ((), ())),
            preferred_element_type=jnp.float32)
        o_ref[j] = jnp.maximum(outj, 0.0)


def kernel(x, lam, gam):
    b, m, f = x.shape
    o = lam.shape[1]
    xt = jnp.transpose(x, (0, 2, 1))
    w = jnp.concatenate([lam, -gam], axis=0)
    bb = _BB
    return pl.pallas_call(
        _eqv_body,
        out_shape=jax.ShapeDtypeStruct((b, m, o), x.dtype),
        grid=(b // bb,),
        in_specs=[
            pl.BlockSpec((bb, f, m), lambda i: (i, 0, 0)),
            pl.BlockSpec((2 * f, o), lambda i: (0, 0)),
        ],
        out_specs=pl.BlockSpec((bb, m, o), lambda i: (i, 0, 0)),
        compiler_params=pltpu.CompilerParams(
            dimension_semantics=("parallel",),
        ),
        name="equivariant_fused",
    )(xt, w)
